# baseline (speedup 1.0000x reference)
import functools

import jax
import jax.numpy as jnp
from jax import lax
from jax.experimental import pallas as pl
from jax.experimental.pallas import tpu as pltpu
from jax.experimental.pallas import tpu_sc as plsc

VOCAB = 1000000
K = 64
N_CLASSES = 2
B = 4096
L = 200

NC = 2
NS = 16
NW = NC * NS
ROWS = B // NW
CHUNKS = ((0, 104), (104, 96))
ACC_UNROLL = 8


def _body(x_hbm, emb_hbm, w_hbm, bt_hbm, out_hbm, idx_v, buf, w_v, b_v,
          out_buf, sem0, sem1):
  sems = (sem0, sem1)
  wid = lax.axis_index("s") * NC + lax.axis_index("c")
  base = wid * ROWS

  pltpu.sync_copy(x_hbm.at[pl.ds(base * L, ROWS * L)], idx_v)
  pltpu.sync_copy(w_hbm, w_v)
  pltpu.sync_copy(bt_hbm, b_v)

  w0 = [w_v[0, pl.ds(16 * j, 16)] for j in range(4)]
  w1 = [w_v[1, pl.ds(16 * j, 16)] for j in range(4)]
  bvec = b_v[...]
  lanes = jnp.arange(16, dtype=jnp.int32)
  inv_l = jnp.float32(1.0 / L)

  def fire(row, slot):
    for off, n in CHUNKS:
      pltpu.async_copy(
          emb_hbm.at[idx_v.at[pl.ds(row * L + off, n)]],
          buf.at[slot, pl.ds(off, n)],
          sems[slot])

  def drain(row, slot):
    for off, n in CHUNKS:
      pltpu.make_async_copy(
          emb_hbm.at[idx_v.at[pl.ds(row * L + off, n)]],
          buf.at[slot, pl.ds(off, n)],
          sems[slot]).wait()

  fire(0, 0)

  def row_step(i, _):
    for sl in range(2):
      row = 2 * i + sl

      @pl.when(row < ROWS - 1)
      def _():
        fire(row + 1, 1 - sl)

      drain(row, sl)

      def acc_step(t, carry):
        accs = list(carry)
        for u in range(ACC_UNROLL):
          l = t * ACC_UNROLL + u
          for j in range(4):
            accs[j] = accs[j] + buf[sl, l, pl.ds(16 * j, 16)]
        return tuple(accs)

      zero = jnp.zeros((16,), jnp.float32)
      a = lax.fori_loop(0, L // ACC_UNROLL, acc_step, (zero, zero, zero, zero))

      t0 = a[0] * w0[0] + a[1] * w0[1] + a[2] * w0[2] + a[3] * w0[3]
      t1 = a[0] * w1[0] + a[1] * w1[1] + a[2] * w1[2] + a[3] * w1[3]
      s0 = jnp.sum(t0)
      s1 = jnp.sum(t1)
      res = jnp.where(lanes == 0, s0, s1) * inv_l + bvec
      plsc.store_scatter(
          out_buf,
          [jnp.full((16,), row, jnp.int32), lanes],
          res,
          mask=lanes < N_CLASSES)
    return 0

  lax.fori_loop(0, ROWS // 2, row_step, 0)

  pltpu.sync_copy(out_buf, out_hbm.at[pl.ds(base, ROWS)])


def _scratch():
  return (
      pltpu.VMEM((ROWS * L,), jnp.int32),
      pltpu.VMEM((2, L, K), jnp.float32),
      pltpu.VMEM((N_CLASSES, K), jnp.float32),
      pltpu.VMEM((16,), jnp.float32),
      pltpu.VMEM((ROWS, N_CLASSES), jnp.float32),
      pltpu.SemaphoreType.DMA,
      pltpu.SemaphoreType.DMA,
  )


@jax.jit
def _run(x, emb, w, bt):
  mesh = plsc.VectorSubcoreMesh(core_axis_name="c", subcore_axis_name="s")
  return pl.kernel(
      _body,
      out_type=jax.ShapeDtypeStruct((B, N_CLASSES), jnp.float32),
      mesh=mesh,
      scratch_types=list(_scratch()),
      compiler_params=pltpu.CompilerParams(
          needs_layout_passes=False, use_tc_tiling_on_sc=False),
  )(x, emb, w, bt)


def kernel(x, emb, W, b):
  bt = jnp.tile(b, 16 // N_CLASSES)
  return _run(x.astype(jnp.int32).reshape(-1), emb, W, bt)


# speedup vs baseline: 3.9974x; 3.9974x over previous
import functools

import jax
import jax.numpy as jnp
from jax import lax
from jax.experimental import pallas as pl
from jax.experimental.pallas import tpu as pltpu
from jax.experimental.pallas import tpu_sc as plsc

VOCAB = 1000000
K = 64
N_CLASSES = 2
B = 4096
L = 200

NC = 2
NS = 16
NW = NC * NS
ROWS = B // NW
NLOOK = ROWS * L
VBLK = 16384
GCHUNK = 1600


def _proj_body(w_ref, embt_ref, p0_ref, p1_ref):
  pt = jnp.dot(w_ref[...], embt_ref[...], preferred_element_type=jnp.float32)
  p0_ref[...] = pt[0]
  p1_ref[...] = pt[1]


def _project(w, embt):
  grid = (VOCAB + VBLK - 1) // VBLK
  return pl.pallas_call(
      _proj_body,
      grid=(grid,),
      in_specs=[
          pl.BlockSpec((N_CLASSES, K), lambda i: (0, 0)),
          pl.BlockSpec((K, VBLK), lambda i: (0, i)),
      ],
      out_specs=[
          pl.BlockSpec((VBLK,), lambda i: (i,)),
          pl.BlockSpec((VBLK,), lambda i: (i,)),
      ],
      out_shape=[
          jax.ShapeDtypeStruct((VOCAB,), jnp.float32),
          jax.ShapeDtypeStruct((VOCAB,), jnp.float32),
      ],
  )(w, embt)


def _pool_body(x_hbm, p0_hbm, p1_hbm, bt_hbm, out_hbm, idx_v, v0, v1, b_v,
               out_buf, sem):
  wid = lax.axis_index("s") * NC + lax.axis_index("c")
  base = wid * ROWS

  pltpu.sync_copy(x_hbm.at[pl.ds(base * L, NLOOK)], idx_v.at[pl.ds(0, NLOOK)])
  pltpu.sync_copy(bt_hbm, b_v)

  for plane_hbm, dst in ((p0_hbm, v0), (p1_hbm, v1)):
    for c in range(NLOOK // GCHUNK):
      pltpu.async_copy(
          plane_hbm.at[idx_v.at[pl.ds(c * GCHUNK, GCHUNK)]],
          dst.at[pl.ds(c * GCHUNK, GCHUNK)],
          sem)
  for plane_hbm, dst in ((p0_hbm, v0), (p1_hbm, v1)):
    for c in range(NLOOK // GCHUNK):
      pltpu.make_async_copy(
          plane_hbm.at[idx_v.at[pl.ds(c * GCHUNK, GCHUNK)]],
          dst.at[pl.ds(c * GCHUNK, GCHUNK)],
          sem).wait()

  bvec = b_v[...]
  lanes = jnp.arange(16, dtype=jnp.int32)
  tail_mask = lanes < 8
  inv_l = jnp.float32(1.0 / L)
  zero = jnp.zeros((16,), jnp.float32)

  def row_step(r, _):
    off = r * L

    def plane_sum(v):
      acc = zero
      for t in range(12):
        acc = acc + v[pl.ds(off + 16 * t, 16)]
      tail = v[pl.ds(off + 192, 16)]
      acc = acc + jnp.where(tail_mask, tail, zero)
      return jnp.sum(acc)

    s0 = plane_sum(v0)
    s1 = plane_sum(v1)
    res = jnp.where(lanes == 0, s0, s1) * inv_l + bvec
    plsc.store_scatter(
        out_buf,
        [lanes, jnp.full((16,), r, jnp.int32)],
        res,
        mask=lanes < N_CLASSES)
    return 0

  lax.fori_loop(0, ROWS, row_step, 0)

  pltpu.sync_copy(out_buf, out_hbm.at[:, pl.ds(base, ROWS)])


def _pool(x, p0, p1, bt):
  mesh = plsc.VectorSubcoreMesh(core_axis_name="c", subcore_axis_name="s")
  return pl.kernel(
      _pool_body,
      out_type=jax.ShapeDtypeStruct((N_CLASSES, B), jnp.float32),
      mesh=mesh,
      scratch_types=[
          pltpu.VMEM((NLOOK,), jnp.int32),
          pltpu.VMEM((NLOOK + 16,), jnp.float32),
          pltpu.VMEM((NLOOK + 16,), jnp.float32),
          pltpu.VMEM((16,), jnp.float32),
          pltpu.VMEM((N_CLASSES, ROWS), jnp.float32),
          pltpu.SemaphoreType.DMA,
      ],
      compiler_params=pltpu.CompilerParams(needs_layout_passes=False),
  )(x, p0, p1, bt)


@jax.jit
def _run(x, embt, w, bt):
  p0, p1 = _project(w, embt)
  return _pool(x, p0, p1, bt).T


def kernel(x, emb, W, b):
  bt = jnp.tile(b, 16 // N_CLASSES)
  return _run(x.astype(jnp.int32).reshape(-1), emb.T, W, bt)


# speedup vs baseline: 4.1259x; 1.0321x over previous
import functools

import jax
import jax.numpy as jnp
from jax import lax
from jax.experimental import pallas as pl
from jax.experimental.pallas import tpu as pltpu
from jax.experimental.pallas import tpu_sc as plsc

VOCAB = 1000000
K = 64
N_CLASSES = 2
B = 4096
L = 200

NC = 2
NS = 16
NW = NC * NS
ROWS = B // NW
VBLK = 16384
DEPTH = 8


def _proj_body(w_ref, embt_ref, p0_ref, p1_ref):
  pt = jnp.dot(w_ref[...], embt_ref[...], preferred_element_type=jnp.float32)
  p0_ref[...] = pt[0]
  p1_ref[...] = pt[1]


def _project(w, embt):
  grid = (VOCAB + VBLK - 1) // VBLK
  return pl.pallas_call(
      _proj_body,
      grid=(grid,),
      in_specs=[
          pl.BlockSpec((N_CLASSES, K), lambda i: (0, 0)),
          pl.BlockSpec((K, VBLK), lambda i: (0, i)),
      ],
      out_specs=[
          pl.BlockSpec((VBLK,), lambda i: (i,)),
          pl.BlockSpec((VBLK,), lambda i: (i,)),
      ],
      out_shape=[
          jax.ShapeDtypeStruct((VOCAB,), jnp.float32),
          jax.ShapeDtypeStruct((VOCAB,), jnp.float32),
      ],
  )(w, embt)


def _pool_body(xt_hbm, p0_hbm, p1_hbm, bt_hbm, out_hbm, xv, v0, v1, b_v, ob,
               *sems):
  wid = lax.axis_index("s") * NC + lax.axis_index("c")
  base = wid * ROWS

  pltpu.sync_copy(xt_hbm.at[:, pl.ds(base, ROWS)], xv)
  pltpu.sync_copy(bt_hbm, b_v)

  planes = ((p0_hbm, v0), (p1_hbm, v1))

  def fire(l, s):
    for plane, v in planes:
      pltpu.async_copy(plane.at[xv.at[l]], v.at[l], sems[s])

  def wait(l, s):
    for plane, v in planes:
      pltpu.make_async_copy(plane.at[xv.at[l]], v.at[l], sems[s]).wait()

  for s in range(DEPTH):
    fire(s, s)

  zero = jnp.zeros((16,), jnp.float32)

  def grp_step(g, accs):
    accs = list(accs)
    for s in range(DEPTH):
      l = g * DEPTH + s
      wait(l, s)

      @pl.when(l + DEPTH < L)
      def _():
        fire(l + DEPTH, s)

      for k in range(8):
        accs[2 * k] = accs[2 * k] + v0[l, pl.ds(16 * k, 16)]
        accs[2 * k + 1] = accs[2 * k + 1] + v1[l, pl.ds(16 * k, 16)]
    return tuple(accs)

  accs = lax.fori_loop(0, L // DEPTH, grp_step, (zero,) * 16)

  b0 = b_v[pl.ds(0, 16)]
  b1 = b_v[pl.ds(16, 16)]
  lanes = jnp.arange(16, dtype=jnp.int32)
  inv_l = jnp.float32(1.0 / L)
  ones = jnp.ones((16,), jnp.bool_)
  for k in range(8):
    pos = 32 * k + 2 * lanes
    plsc.store_scatter(ob, [pos], accs[2 * k] * inv_l + b0, mask=ones)
    plsc.store_scatter(ob, [pos + 1], accs[2 * k + 1] * inv_l + b1, mask=ones)

  pltpu.sync_copy(ob, out_hbm.at[pl.ds(base * N_CLASSES, ROWS * N_CLASSES)])


def _pool(xt, p0, p1, bt):
  mesh = plsc.VectorSubcoreMesh(core_axis_name="c", subcore_axis_name="s")
  return pl.kernel(
      _pool_body,
      out_type=jax.ShapeDtypeStruct((B * N_CLASSES,), jnp.float32),
      mesh=mesh,
      scratch_types=[
          pltpu.VMEM((L, ROWS), jnp.int32),
          pltpu.VMEM((L, ROWS), jnp.float32),
          pltpu.VMEM((L, ROWS), jnp.float32),
          pltpu.VMEM((2 * 16,), jnp.float32),
          pltpu.VMEM((ROWS * N_CLASSES,), jnp.float32),
      ] + [pltpu.SemaphoreType.DMA] * DEPTH,
      compiler_params=pltpu.CompilerParams(
          needs_layout_passes=False, use_tc_tiling_on_sc=False),
  )(xt, p0, p1, bt)


@jax.jit
def _run(xt, embt, w, bt):
  p0, p1 = _project(w, embt)
  return _pool(xt, p0, p1, bt).reshape(B, N_CLASSES)


def kernel(x, emb, W, b):
  bt = jnp.repeat(b, 16)
  return _run(x.astype(jnp.int32).T, emb.T, W, bt)


# speedup vs baseline: 4.9864x; 1.2086x over previous
import functools

import jax
import jax.numpy as jnp
from jax import lax
from jax.experimental import pallas as pl
from jax.experimental.pallas import tpu as pltpu
from jax.experimental.pallas import tpu_sc as plsc

VOCAB = 1000000
K = 64
N_CLASSES = 2
B = 4096
L = 200

NC = 2
NS = 16
NW = NC * NS
ROWS = B // NW
VBLK = 16384
DEPTH = 8


def _proj_body(w_ref, embt_ref, pp_ref):
  pt = jnp.dot(w_ref[...], embt_ref[...], preferred_element_type=jnp.float32)
  lo = lax.bitcast_convert_type(
      pt[0].astype(jnp.bfloat16), jnp.uint16).astype(jnp.uint32)
  hi = lax.bitcast_convert_type(
      pt[1].astype(jnp.bfloat16), jnp.uint16).astype(jnp.uint32)
  pp_ref[...] = lo | (hi << 16)


def _project(w, embt):
  grid = (VOCAB + VBLK - 1) // VBLK
  return pl.pallas_call(
      _proj_body,
      grid=(grid,),
      in_specs=[
          pl.BlockSpec((N_CLASSES, K), lambda i: (0, 0)),
          pl.BlockSpec((K, VBLK), lambda i: (0, i)),
      ],
      out_specs=pl.BlockSpec((VBLK,), lambda i: (i,)),
      out_shape=jax.ShapeDtypeStruct((VOCAB,), jnp.uint32),
  )(w, embt)


def _pool_body(xt_hbm, pp_hbm, bt_hbm, out_hbm, xv, v, b_v, ob, *sems):
  wid = lax.axis_index("s") * NC + lax.axis_index("c")
  base = wid * ROWS

  pltpu.sync_copy(xt_hbm.at[:, pl.ds(base, ROWS)], xv)
  pltpu.sync_copy(bt_hbm, b_v)

  def fire(l, s):
    pltpu.async_copy(pp_hbm.at[xv.at[l]], v.at[s], sems[s])

  def wait(l, s):
    pltpu.make_async_copy(pp_hbm.at[xv.at[l]], v.at[s], sems[s]).wait()

  for s in range(DEPTH):
    fire(s, s)

  zero = jnp.zeros((16,), jnp.float32)
  himask = jnp.full((16,), 0xFFFF0000, jnp.uint32)

  def grp_step(g, accs):
    accs = list(accs)
    for s in range(DEPTH):
      l = g * DEPTH + s
      wait(l, s)
      vals = []
      for k in range(8):
        packed = v[s, pl.ds(16 * k, 16)]
        vals.append(lax.bitcast_convert_type(packed << 16, jnp.float32))
        vals.append(lax.bitcast_convert_type(packed & himask, jnp.float32))

      @pl.when(l + DEPTH < L)
      def _():
        fire(l + DEPTH, s)

      for k in range(16):
        accs[k] = accs[k] + vals[k]
    return tuple(accs)

  accs = lax.fori_loop(0, L // DEPTH, grp_step, (zero,) * 16)

  b0 = b_v[pl.ds(0, 16)]
  b1 = b_v[pl.ds(16, 16)]
  lanes = jnp.arange(16, dtype=jnp.int32)
  inv_l = jnp.float32(1.0 / L)
  ones = jnp.ones((16,), jnp.bool_)
  for k in range(8):
    pos = 32 * k + 2 * lanes
    plsc.store_scatter(ob, [pos], accs[2 * k] * inv_l + b0, mask=ones)
    plsc.store_scatter(ob, [pos + 1], accs[2 * k + 1] * inv_l + b1, mask=ones)

  pltpu.sync_copy(ob, out_hbm.at[pl.ds(base * N_CLASSES, ROWS * N_CLASSES)])


def _pool(xt, pp, bt):
  mesh = plsc.VectorSubcoreMesh(core_axis_name="c", subcore_axis_name="s")
  return pl.kernel(
      _pool_body,
      out_type=jax.ShapeDtypeStruct((B * N_CLASSES,), jnp.float32),
      mesh=mesh,
      scratch_types=[
          pltpu.VMEM((L, ROWS), jnp.int32),
          pltpu.VMEM((DEPTH, ROWS), jnp.uint32),
          pltpu.VMEM((2 * 16,), jnp.float32),
          pltpu.VMEM((ROWS * N_CLASSES,), jnp.float32),
      ] + [pltpu.SemaphoreType.DMA] * DEPTH,
      compiler_params=pltpu.CompilerParams(
          needs_layout_passes=False, use_tc_tiling_on_sc=False),
  )(xt, pp, bt)


@jax.jit
def _run(xt, embt, w, bt):
  pp = _project(w, embt)
  return _pool(xt, pp, bt).reshape(B, N_CLASSES)


def kernel(x, emb, W, b):
  bt = jnp.repeat(b, 16)
  return _run(x.astype(jnp.int32).T, emb.T, W, bt)


# speedup vs baseline: 5.3557x; 1.0741x over previous
import functools

import jax
import jax.numpy as jnp
from jax import lax
from jax.experimental import pallas as pl
from jax.experimental.pallas import tpu as pltpu
from jax.experimental.pallas import tpu_sc as plsc

VOCAB = 1000000
K = 64
N_CLASSES = 2
B = 4096
L = 200

NC = 2
NS = 16
NW = NC * NS
ROWS = B // NW
VBLK = 32768
DEPTH = 8


def _proj_body(w_ref, embt_ref, pp_ref):
  pt = jnp.dot(w_ref[...], embt_ref[...], preferred_element_type=jnp.float32)
  lo = lax.bitcast_convert_type(
      pt[0].astype(jnp.bfloat16), jnp.uint16).astype(jnp.uint32)
  hi = lax.bitcast_convert_type(
      pt[1].astype(jnp.bfloat16), jnp.uint16).astype(jnp.uint32)
  pp_ref[...] = lo | (hi << 16)


def _project(w, embt):
  grid = (VOCAB + VBLK - 1) // VBLK
  return pl.pallas_call(
      _proj_body,
      grid=(grid,),
      in_specs=[
          pl.BlockSpec((N_CLASSES, K), lambda i: (0, 0)),
          pl.BlockSpec((K, VBLK), lambda i: (0, i)),
      ],
      out_specs=pl.BlockSpec((VBLK,), lambda i: (i,)),
      out_shape=jax.ShapeDtypeStruct((VOCAB,), jnp.uint32),
  )(w, embt)


def _pool_body(xt_hbm, pp_hbm, bt_hbm, out_hbm, xv, v, b_v, ob, *sems):
  wid = lax.axis_index("s") * NC + lax.axis_index("c")
  base = wid * ROWS

  pltpu.sync_copy(xt_hbm.at[:, pl.ds(base, ROWS)], xv)
  pltpu.sync_copy(bt_hbm, b_v)

  def fire(l, s):
    pltpu.async_copy(pp_hbm.at[xv.at[l]], v.at[s], sems[s])

  def wait(l, s):
    pltpu.make_async_copy(pp_hbm.at[xv.at[l]], v.at[s], sems[s]).wait()

  for s in range(DEPTH):
    fire(s, s)

  zero = jnp.zeros((16,), jnp.float32)
  himask = jnp.full((16,), 0xFFFF0000, jnp.uint32)

  def grp_step(g, accs):
    accs = list(accs)
    for s in range(DEPTH):
      l = g * DEPTH + s
      wait(l, s)
      vals = []
      for k in range(8):
        packed = v[s, pl.ds(16 * k, 16)]
        vals.append(lax.bitcast_convert_type(packed << 16, jnp.float32))
        vals.append(lax.bitcast_convert_type(packed & himask, jnp.float32))

      @pl.when(l + DEPTH < L)
      def _():
        fire(l + DEPTH, s)

      for k in range(16):
        accs[k] = accs[k] + vals[k]
    return tuple(accs)

  accs = lax.fori_loop(0, L // DEPTH, grp_step, (zero,) * 16)

  b0 = b_v[pl.ds(0, 16)]
  b1 = b_v[pl.ds(16, 16)]
  lanes = jnp.arange(16, dtype=jnp.int32)
  inv_l = jnp.float32(1.0 / L)
  ones = jnp.ones((16,), jnp.bool_)
  for k in range(8):
    pos = 32 * k + 2 * lanes
    plsc.store_scatter(ob, [pos], accs[2 * k] * inv_l + b0, mask=ones)
    plsc.store_scatter(ob, [pos + 1], accs[2 * k + 1] * inv_l + b1, mask=ones)

  pltpu.sync_copy(ob, out_hbm.at[pl.ds(base * N_CLASSES, ROWS * N_CLASSES)])


def _pool(xt, pp, bt):
  mesh = plsc.VectorSubcoreMesh(core_axis_name="c", subcore_axis_name="s")
  return pl.kernel(
      _pool_body,
      out_type=jax.ShapeDtypeStruct((B * N_CLASSES,), jnp.float32),
      mesh=mesh,
      scratch_types=[
          pltpu.VMEM((L, ROWS), jnp.int32),
          pltpu.VMEM((DEPTH, ROWS), jnp.uint32),
          pltpu.VMEM((2 * 16,), jnp.float32),
          pltpu.VMEM((ROWS * N_CLASSES,), jnp.float32),
      ] + [pltpu.SemaphoreType.DMA] * DEPTH,
      compiler_params=pltpu.CompilerParams(
          needs_layout_passes=False, use_tc_tiling_on_sc=False),
  )(xt, pp, bt)


@jax.jit
def _run(xt, embt, w, bt):
  pp = _project(w, embt)
  return _pool(xt, pp, bt).reshape(B, N_CLASSES)


def kernel(x, emb, W, b):
  bt = jnp.repeat(b, 16)
  return _run(x.astype(jnp.int32).T, emb.T, W, bt)
